# Initial kernel scaffold; baseline (speedup 1.0000x reference)
import jax
import jax.numpy as jnp
from jax.experimental import pallas as pl


def kernel(edge_index, x, W1_rel, b1_rel, W1_root, W2_rel, b2_rel, W2_root, W3_rel, b3_rel, W3_root):
    raise NotImplementedError("write your pallas kernel here")


# speedup vs baseline: 24.6190x; 24.6190x over previous
import functools

import jax
import jax.numpy as jnp
from jax import lax
from jax.experimental import pallas as pl
from jax.experimental.pallas import tpu as pltpu
from jax.experimental.pallas import tpu_sc as plsc

N_NODES = 10000
N_EDGES = 160000
NW = 32
LANES = 16
E_PER_W = N_EDGES // NW
FULL_GROUPS = E_PER_W // LANES
TAIL = E_PER_W - FULL_GROUPS * LANES
EBUF = FULL_GROUPS * LANES + LANES


def _make_edge_agg(feat):
    mesh = plsc.VectorSubcoreMesh(core_axis_name="c", subcore_axis_name="s")

    @functools.partial(
        pl.kernel,
        out_type=jax.ShapeDtypeStruct((NW, feat, N_NODES), jnp.float32),
        mesh=mesh,
        compiler_params=pltpu.CompilerParams(needs_layout_passes=False),
        scratch_types=[
            pltpu.VMEM((feat, N_NODES), jnp.float32),
            pltpu.VMEM((feat, N_NODES), jnp.float32),
            pltpu.VMEM((EBUF,), jnp.int32),
            pltpu.VMEM((EBUF,), jnp.int32),
        ],
    )
    def edge_agg(p_hbm, src_hbm, dst_hbm, out_hbm, p_v, agg_v, src_v, dst_v):
        wid = lax.axis_index("s") * 2 + lax.axis_index("c")
        base = wid * E_PER_W
        pltpu.sync_copy(p_hbm, p_v)
        pltpu.sync_copy(src_hbm.at[pl.ds(base, E_PER_W)],
                        src_v.at[pl.ds(0, E_PER_W)])
        pltpu.sync_copy(dst_hbm.at[pl.ds(base, E_PER_W)],
                        dst_v.at[pl.ds(0, E_PER_W)])

        zeros = jnp.zeros((LANES,), jnp.float32)

        def zero_body(i, carry):
            for f in range(feat):
                agg_v[f, pl.ds(i * LANES, LANES)] = zeros
            return carry

        lax.fori_loop(0, N_NODES // LANES, zero_body, 0)

        rows = [jnp.full((LANES,), f, jnp.int32) for f in range(feat)]

        def edge_body(i, carry):
            s = src_v[pl.ds(i * LANES, LANES)]
            d = dst_v[pl.ds(i * LANES, LANES)]
            for f in range(feat):
                vals = plsc.load_gather(p_v, [rows[f], s])
                plsc.addupdate_scatter(agg_v, [rows[f], d], vals)
            return carry

        lax.fori_loop(0, FULL_GROUPS, edge_body, 0)

        mask = lax.iota(jnp.int32, LANES) < TAIL
        s = jnp.where(mask, src_v[pl.ds(FULL_GROUPS * LANES, LANES)], 0)
        d = jnp.where(mask, dst_v[pl.ds(FULL_GROUPS * LANES, LANES)], 0)
        for f in range(feat):
            vals = plsc.load_gather(p_v, [rows[f], s])
            plsc.addupdate_scatter(agg_v, [rows[f], d], vals, mask=mask)

        pltpu.sync_copy(agg_v, out_hbm.at[wid])

    return edge_agg


_edge_agg_f4 = _make_edge_agg(4)
_edge_agg_f2 = _make_edge_agg(2)


def _proj_kernel(x_ref, w_ref, b_ref, p_ref, r_ref, *, split):
    res = lax.dot_general(w_ref[...], x_ref[...], (((1,), (1,)), ((), ())),
                          preferred_element_type=jnp.float32)
    res = res + b_ref[...]
    p_ref[...] = res[:split, :]
    r_ref[...] = res[split:, :]


def _project(x, w_cat_t, b_cat, split):
    fc = w_cat_t.shape[0]
    return pl.pallas_call(
        functools.partial(_proj_kernel, split=split),
        out_shape=(
            jax.ShapeDtypeStruct((split, N_NODES), jnp.float32),
            jax.ShapeDtypeStruct((fc - split, N_NODES), jnp.float32),
        ),
    )(x, w_cat_t, b_cat.reshape(fc, 1))


def _combine_proj_kernel(parts_ref, r_ref, w_ref, b_ref, p_ref, rn_ref, *,
                         split):
    h = jnp.tanh(jnp.sum(parts_ref[...], axis=0) + r_ref[...])
    res = lax.dot_general(w_ref[...], h, (((1,), (0,)), ((), ())),
                          preferred_element_type=jnp.float32)
    res = res + b_ref[...]
    p_ref[...] = res[:split, :]
    rn_ref[...] = res[split:, :]


def _combine_project(partials, r, w_cat_t, b_cat, split):
    fc = w_cat_t.shape[0]
    return pl.pallas_call(
        functools.partial(_combine_proj_kernel, split=split),
        out_shape=(
            jax.ShapeDtypeStruct((split, N_NODES), jnp.float32),
            jax.ShapeDtypeStruct((fc - split, N_NODES), jnp.float32),
        ),
    )(partials, r, w_cat_t, b_cat.reshape(fc, 1))


def _finish_kernel(parts_ref, r_ref, out_ref):
    out_ref[...] = jnp.tanh(jnp.sum(parts_ref[...], axis=0) + r_ref[...])


def _finish(partials, r):
    return pl.pallas_call(
        _finish_kernel,
        out_shape=jax.ShapeDtypeStruct(r.shape, jnp.float32),
    )(partials, r)


def kernel(edge_index, x, W1_rel, b1_rel, W1_root, W2_rel, b2_rel, W2_root,
           W3_rel, b3_rel, W3_root):
    src = edge_index[0]
    dst = edge_index[1]

    w1t = jnp.concatenate([W1_rel, W1_root], axis=1).T
    b1 = jnp.concatenate([jnp.zeros((4,), jnp.float32), b1_rel])
    w2t = jnp.concatenate([W2_rel, W2_root], axis=1).T
    b2 = jnp.concatenate([jnp.zeros((4,), jnp.float32), b2_rel])
    w3t = jnp.concatenate([W3_rel, W3_root], axis=1).T
    b3 = jnp.concatenate([jnp.zeros((2,), jnp.float32), b3_rel])

    p1, r1 = _project(x, w1t, b1, 4)
    parts1 = _edge_agg_f4(p1, src, dst)
    p2, r2 = _combine_project(parts1, r1, w2t, b2, 4)
    parts2 = _edge_agg_f4(p2, src, dst)
    p3, r3 = _combine_project(parts2, r2, w3t, b3, 2)
    parts3 = _edge_agg_f2(p3, src, dst)
    return _finish(parts3, r3).T


# speedup vs baseline: 27.1237x; 1.1017x over previous
import functools

import jax
import jax.numpy as jnp
from jax import lax
from jax.experimental import pallas as pl
from jax.experimental.pallas import tpu as pltpu
from jax.experimental.pallas import tpu_sc as plsc

N_NODES = 10000
N_EDGES = 160000
NW = 32
LANES = 16
E_PER_W = N_EDGES // NW
FULL_GROUPS = E_PER_W // LANES
TAIL = E_PER_W - FULL_GROUPS * LANES
EBUF = FULL_GROUPS * LANES + LANES


def _make_edge_agg(feat):
    mesh = plsc.VectorSubcoreMesh(core_axis_name="c", subcore_axis_name="s")

    @functools.partial(
        pl.kernel,
        out_type=jax.ShapeDtypeStruct((NW * feat, N_NODES), jnp.float32),
        mesh=mesh,
        compiler_params=pltpu.CompilerParams(needs_layout_passes=False),
        scratch_types=[
            pltpu.VMEM((feat, N_NODES), jnp.float32),
            pltpu.VMEM((feat, N_NODES), jnp.float32),
            pltpu.VMEM((EBUF,), jnp.int32),
            pltpu.VMEM((EBUF,), jnp.int32),
            pltpu.SemaphoreType.DMA,
            pltpu.SemaphoreType.DMA,
            pltpu.SemaphoreType.DMA,
        ],
    )
    def edge_agg(p_hbm, src_hbm, dst_hbm, out_hbm, p_v, agg_v, src_v, dst_v,
                 sem_p, sem_s, sem_d):
        wid = lax.axis_index("s") * 2 + lax.axis_index("c")
        base = wid * E_PER_W
        cp_p = pltpu.async_copy(p_hbm, p_v, sem_p)
        cp_s = pltpu.async_copy(src_hbm.at[pl.ds(base, E_PER_W)],
                                src_v.at[pl.ds(0, E_PER_W)], sem_s)
        cp_d = pltpu.async_copy(dst_hbm.at[pl.ds(base, E_PER_W)],
                                dst_v.at[pl.ds(0, E_PER_W)], sem_d)

        zeros = jnp.zeros((LANES,), jnp.float32)

        def zero_body(i, carry):
            for f in range(feat):
                agg_v[f, pl.ds(i * LANES, LANES)] = zeros
            return carry

        lax.fori_loop(0, N_NODES // LANES, zero_body, 0)
        cp_p.wait()
        cp_s.wait()
        cp_d.wait()

        rows = [jnp.full((LANES,), f, jnp.int32) for f in range(feat)]

        def edge_body(i, carry):
            s = src_v[pl.ds(i * LANES, LANES)]
            d = dst_v[pl.ds(i * LANES, LANES)]
            for f in range(feat):
                vals = plsc.load_gather(p_v, [rows[f], s])
                plsc.addupdate_scatter(agg_v, [rows[f], d], vals)
            return carry

        lax.fori_loop(0, FULL_GROUPS, edge_body, 0)

        mask = lax.iota(jnp.int32, LANES) < TAIL
        s = jnp.where(mask, src_v[pl.ds(FULL_GROUPS * LANES, LANES)], 0)
        d = jnp.where(mask, dst_v[pl.ds(FULL_GROUPS * LANES, LANES)], 0)
        for f in range(feat):
            vals = plsc.load_gather(p_v, [rows[f], s])
            plsc.addupdate_scatter(agg_v, [rows[f], d], vals, mask=mask)

        pltpu.sync_copy(agg_v, out_hbm.at[pl.ds(wid * feat, feat)])

    return edge_agg


_edge_agg_f4 = _make_edge_agg(4)
_edge_agg_f2 = _make_edge_agg(2)


def _proj_kernel(x_ref, w_ref, b_ref, p_ref, r_ref, *, split):
    res = lax.dot_general(w_ref[...], x_ref[...], (((1,), (1,)), ((), ())),
                          preferred_element_type=jnp.float32)
    res = res + b_ref[...]
    p_ref[...] = res[:split, :]
    r_ref[...] = res[split:, :]


def _project(x, w_cat_t, b_cat, split):
    fc = w_cat_t.shape[0]
    return pl.pallas_call(
        functools.partial(_proj_kernel, split=split),
        out_shape=(
            jax.ShapeDtypeStruct((split, N_NODES), jnp.float32),
            jax.ShapeDtypeStruct((fc - split, N_NODES), jnp.float32),
        ),
    )(x, w_cat_t, b_cat.reshape(fc, 1))


def _combine_proj_kernel(parts_ref, r_ref, w_ref, b_ref, p_ref, rn_ref, *,
                         split):
    feat = r_ref.shape[0]
    parts = parts_ref[...].reshape(NW, feat, N_NODES)
    h = jnp.tanh(jnp.sum(parts, axis=0) + r_ref[...])
    res = lax.dot_general(w_ref[...], h, (((1,), (0,)), ((), ())),
                          preferred_element_type=jnp.float32)
    res = res + b_ref[...]
    p_ref[...] = res[:split, :]
    rn_ref[...] = res[split:, :]


def _combine_project(partials, r, w_cat_t, b_cat, split):
    fc = w_cat_t.shape[0]
    return pl.pallas_call(
        functools.partial(_combine_proj_kernel, split=split),
        out_shape=(
            jax.ShapeDtypeStruct((split, N_NODES), jnp.float32),
            jax.ShapeDtypeStruct((fc - split, N_NODES), jnp.float32),
        ),
    )(partials, r, w_cat_t, b_cat.reshape(fc, 1))


def _finish_kernel(parts_ref, r_ref, out_ref):
    feat = r_ref.shape[0]
    parts = parts_ref[...].reshape(NW, feat, N_NODES)
    out_ref[...] = jnp.tanh(jnp.sum(parts, axis=0) + r_ref[...])


def _finish(partials, r):
    return pl.pallas_call(
        _finish_kernel,
        out_shape=jax.ShapeDtypeStruct(r.shape, jnp.float32),
    )(partials, r)


def kernel(edge_index, x, W1_rel, b1_rel, W1_root, W2_rel, b2_rel, W2_root,
           W3_rel, b3_rel, W3_root):
    src = edge_index[0]
    dst = edge_index[1]

    w1t = jnp.concatenate([W1_rel, W1_root], axis=1).T
    b1 = jnp.concatenate([jnp.zeros((4,), jnp.float32), b1_rel])
    w2t = jnp.concatenate([W2_rel, W2_root], axis=1).T
    b2 = jnp.concatenate([jnp.zeros((4,), jnp.float32), b2_rel])
    w3t = jnp.concatenate([W3_rel, W3_root], axis=1).T
    b3 = jnp.concatenate([jnp.zeros((2,), jnp.float32), b3_rel])

    p1, r1 = _project(x, w1t, b1, 4)
    parts1 = _edge_agg_f4(p1, src, dst)
    p2, r2 = _combine_project(parts1, r1, w2t, b2, 4)
    parts2 = _edge_agg_f4(p2, src, dst)
    p3, r3 = _combine_project(parts2, r2, w3t, b3, 2)
    parts3 = _edge_agg_f2(p3, src, dst)
    return _finish(parts3, r3).T


# speedup vs baseline: 28.2032x; 1.0398x over previous
import functools

import jax
import jax.numpy as jnp
from jax import lax
from jax.experimental import pallas as pl
from jax.experimental.pallas import tpu as pltpu
from jax.experimental.pallas import tpu_sc as plsc

N_NODES = 10000
N_EDGES = 160000
NW = 32
LANES = 16
E_PER_W = N_EDGES // NW
FULL_GROUPS = E_PER_W // LANES
TAIL = E_PER_W - FULL_GROUPS * LANES
EBUF = FULL_GROUPS * LANES + LANES


def _make_edge_agg(feat):
    mesh = plsc.VectorSubcoreMesh(core_axis_name="c", subcore_axis_name="s")

    @functools.partial(
        pl.kernel,
        out_type=jax.ShapeDtypeStruct((NW * feat, N_NODES), jnp.float32),
        mesh=mesh,
        compiler_params=pltpu.CompilerParams(needs_layout_passes=False),
        scratch_types=[
            pltpu.VMEM((feat, N_NODES), jnp.float32),
            pltpu.VMEM((feat, N_NODES), jnp.float32),
            pltpu.VMEM((EBUF,), jnp.int32),
            pltpu.VMEM((EBUF,), jnp.int32),
            pltpu.SemaphoreType.DMA,
            pltpu.SemaphoreType.DMA,
            pltpu.SemaphoreType.DMA,
        ],
    )
    def edge_agg(p_hbm, ei_hbm, out_hbm, p_v, agg_v, src_v, dst_v,
                 sem_p, sem_s, sem_d):
        wid = lax.axis_index("s") * 2 + lax.axis_index("c")
        base = wid * E_PER_W
        cp_p = pltpu.async_copy(p_hbm, p_v, sem_p)
        cp_s = pltpu.async_copy(ei_hbm.at[pl.ds(base, E_PER_W)],
                                src_v.at[pl.ds(0, E_PER_W)], sem_s)
        cp_d = pltpu.async_copy(ei_hbm.at[pl.ds(N_EDGES + base, E_PER_W)],
                                dst_v.at[pl.ds(0, E_PER_W)], sem_d)

        zeros = jnp.zeros((LANES,), jnp.float32)

        def zero_body(i, carry):
            for f in range(feat):
                agg_v[f, pl.ds(i * LANES, LANES)] = zeros
            return carry

        lax.fori_loop(0, N_NODES // LANES, zero_body, 0)
        cp_p.wait()
        cp_s.wait()
        cp_d.wait()

        rows = [jnp.full((LANES,), f, jnp.int32) for f in range(feat)]

        def edge_body(i, carry):
            s = src_v[pl.ds(i * LANES, LANES)]
            d = dst_v[pl.ds(i * LANES, LANES)]
            for f in range(feat):
                vals = plsc.load_gather(p_v, [rows[f], s])
                plsc.addupdate_scatter(agg_v, [rows[f], d], vals)
            return carry

        lax.fori_loop(0, FULL_GROUPS, edge_body, 0)

        mask = lax.iota(jnp.int32, LANES) < TAIL
        s = jnp.where(mask, src_v[pl.ds(FULL_GROUPS * LANES, LANES)], 0)
        d = jnp.where(mask, dst_v[pl.ds(FULL_GROUPS * LANES, LANES)], 0)
        for f in range(feat):
            vals = plsc.load_gather(p_v, [rows[f], s])
            plsc.addupdate_scatter(agg_v, [rows[f], d], vals, mask=mask)

        pltpu.sync_copy(agg_v, out_hbm.at[pl.ds(wid * feat, feat)])

    return edge_agg


_edge_agg_f4 = _make_edge_agg(4)
_edge_agg_f2 = _make_edge_agg(2)


def _bias_rows(b, shape):
    return lax.broadcast_in_dim(b, shape, (0,))


def _proj_kernel(x_ref, wr_ref, wo_ref, b_ref, p_ref, r_ref):
    x = x_ref[...]
    p_ref[...] = lax.dot_general(wr_ref[...], x, (((0,), (1,)), ((), ())),
                                 preferred_element_type=jnp.float32)
    r = lax.dot_general(wo_ref[...], x, (((0,), (1,)), ((), ())),
                        preferred_element_type=jnp.float32)
    r_ref[...] = r + _bias_rows(b_ref[...], r.shape)


def _project(x, w_rel, w_root, b):
    split = w_rel.shape[1]
    return pl.pallas_call(
        _proj_kernel,
        out_shape=(
            jax.ShapeDtypeStruct((split, N_NODES), jnp.float32),
            jax.ShapeDtypeStruct((w_root.shape[1], N_NODES), jnp.float32),
        ),
    )(x, w_rel, w_root, b)


def _combine_proj_kernel(parts_ref, r_ref, wr_ref, wo_ref, b_ref,
                         p_ref, rn_ref):
    feat = r_ref.shape[0]
    parts = parts_ref[...].reshape(NW, feat, N_NODES)
    h = jnp.tanh(jnp.sum(parts, axis=0) + r_ref[...])
    p_ref[...] = lax.dot_general(wr_ref[...], h, (((0,), (0,)), ((), ())),
                                 preferred_element_type=jnp.float32)
    rn = lax.dot_general(wo_ref[...], h, (((0,), (0,)), ((), ())),
                         preferred_element_type=jnp.float32)
    rn_ref[...] = rn + _bias_rows(b_ref[...], rn.shape)


def _combine_project(partials, r, w_rel, w_root, b):
    return pl.pallas_call(
        _combine_proj_kernel,
        out_shape=(
            jax.ShapeDtypeStruct((w_rel.shape[1], N_NODES), jnp.float32),
            jax.ShapeDtypeStruct((w_root.shape[1], N_NODES), jnp.float32),
        ),
    )(partials, r, w_rel, w_root, b)


def _finish_kernel(parts_ref, r_ref, out_ref):
    feat = r_ref.shape[0]
    parts = parts_ref[...].reshape(NW, feat, N_NODES)
    out_ref[...] = jnp.tanh(jnp.sum(parts, axis=0) + r_ref[...])


def _finish(partials, r):
    return pl.pallas_call(
        _finish_kernel,
        out_shape=jax.ShapeDtypeStruct(r.shape, jnp.float32),
    )(partials, r)


def kernel(edge_index, x, W1_rel, b1_rel, W1_root, W2_rel, b2_rel, W2_root,
           W3_rel, b3_rel, W3_root):
    ei_flat = edge_index.reshape(-1)
    p1, r1 = _project(x, W1_rel, W1_root, b1_rel)
    parts1 = _edge_agg_f4(p1, ei_flat)
    p2, r2 = _combine_project(parts1, r1, W2_rel, W2_root, b2_rel)
    parts2 = _edge_agg_f4(p2, ei_flat)
    p3, r3 = _combine_project(parts2, r2, W3_rel, W3_root, b3_rel)
    parts3 = _edge_agg_f2(p3, ei_flat)
    return _finish(parts3, r3).T
